# baseline (speedup 1.0000x reference)
import functools

import numpy as np
import jax
import jax.numpy as jnp
from jax import lax
from jax.experimental import pallas as pl
from jax.experimental.pallas import tpu as pltpu
from jax.experimental.pallas import tpu_sc as plsc

B = 16384
F = 26
V = 100000
D = 32
ND = 13
NF = F + ND

NC = 2
NS = 16
NW = NC * NS

IDXW = 128
ROWS_W = B * F // (IDXW * NW)
DROWS_W = B * ND // (IDXW * NW)
K = 8
GROUPS = ROWS_W // K
DK = 4
DGROUPS = DROWS_W // DK
BPW = B // NW


def _consts():
    p = np.arange(B * F, dtype=np.int64)
    oidx_s = (p // F) * NF + (p % F)
    q = np.arange(B * ND, dtype=np.int64)
    oidx_d = (q // ND) * NF + F + (q % ND)
    r = np.arange(13 * IDXW, dtype=np.int64)
    offs = (r % F) * V
    return (oidx_s.astype(np.int32).reshape(-1, IDXW),
            oidx_d.astype(np.int32).reshape(NW, DROWS_W, IDXW),
            offs.astype(np.int32).reshape(13, IDXW))


_OIDX_S, _OIDX_D, _OFFS = _consts()


def _mm_body(x_ref, w_ref, o_ref):
    o_ref[...] = jnp.dot(x_ref[...], w_ref[...],
                         preferred_element_type=jnp.float32)


def _dense_proj(x, w):
    bb = 2048
    return pl.pallas_call(
        _mm_body,
        grid=(B // bb,),
        in_specs=[pl.BlockSpec((bb, ND), lambda i: (i, 0)),
                  pl.BlockSpec((ND, ND * D), lambda i: (0, 0))],
        out_specs=pl.BlockSpec((bb, ND * D), lambda i: (i, 0)),
        out_shape=jax.ShapeDtypeStruct((B, ND * D), jnp.float32),
    )(x, w)


def _sc_body(tbl_hbm, sidx_hbm, dtok_hbm, oidxs_hbm, oidxd_hbm, offs_hbm,
             out_hbm, idxv, oidxv, doidxv, offsv, gbuf, gsem, ssem):
    wid = lax.axis_index("s") * NC + lax.axis_index("c")
    rbase = wid * ROWS_W
    drbase = wid * DROWS_W

    pltpu.sync_copy(sidx_hbm.at[pl.ds(rbase, ROWS_W)], idxv)
    pltpu.sync_copy(oidxs_hbm.at[pl.ds(rbase, ROWS_W)], oidxv)
    pltpu.sync_copy(oidxd_hbm.at[wid], doidxv)
    pltpu.sync_copy(offs_hbm, offsv)

    def add_body(r, carry):
        fr = lax.rem(r, 13)
        for v in range(IDXW // 16):
            sl = pl.ds(v * 16, 16)
            idxv[r, sl] = idxv[r, sl] + offsv[fr, sl]
        return carry
    lax.fori_loop(0, ROWS_W, add_body, 0)

    def group_body(g, carry):
        buf = lax.rem(g, 2) * (K * IDXW)
        b0 = g * K
        for j in range(K):
            pltpu.async_copy(tbl_hbm.at[idxv.at[b0 + j]],
                             gbuf.at[pl.ds(buf + j * IDXW, IDXW)], gsem)
        pltpu.make_async_copy(tbl_hbm.at[pl.ds(0, K * IDXW)],
                              gbuf.at[pl.ds(buf, K * IDXW)], gsem).wait()
        for j in range(K):
            pltpu.async_copy(gbuf.at[pl.ds(buf + j * IDXW, IDXW)],
                             out_hbm.at[oidxv.at[b0 + j]], ssem)
        pltpu.make_async_copy(tbl_hbm.at[pl.ds(0, K * IDXW)],
                              gbuf.at[pl.ds(buf, K * IDXW)], ssem).wait()
        return carry
    lax.fori_loop(0, GROUPS, group_body, 0)

    def dgroup_body(g, carry):
        b0 = g * DK
        for j in range(DK):
            src = dtok_hbm.at[pl.ds((drbase + b0 + j) * IDXW, IDXW)]
            pltpu.async_copy(src, gbuf.at[pl.ds(j * IDXW, IDXW)], gsem)
        pltpu.make_async_copy(tbl_hbm.at[pl.ds(0, DK * IDXW)],
                              gbuf.at[pl.ds(0, DK * IDXW)], gsem).wait()
        for j in range(DK):
            pltpu.async_copy(gbuf.at[pl.ds(j * IDXW, IDXW)],
                             out_hbm.at[doidxv.at[b0 + j]], ssem)
        pltpu.make_async_copy(tbl_hbm.at[pl.ds(0, DK * IDXW)],
                              gbuf.at[pl.ds(0, DK * IDXW)], ssem).wait()
        return carry
    lax.fori_loop(0, DGROUPS, dgroup_body, 0)


@functools.partial(jax.jit, static_argnames=())
def _tokenize(sparse_inputs, dense_inputs, tables, w):
    tbl = tables.reshape(F * V, D)
    sidx = sparse_inputs.reshape(ROWS_W * NW, IDXW)
    dtok = _dense_proj(dense_inputs, w).reshape(B * ND, D)

    mesh = plsc.VectorSubcoreMesh(core_axis_name="c", subcore_axis_name="s",
                                  num_cores=NC, num_subcores=NS)
    sc = pl.kernel(
        _sc_body,
        out_type=jax.ShapeDtypeStruct((B * NF, D), jnp.float32),
        mesh=mesh,
        scratch_types=[
            pltpu.VMEM((ROWS_W, IDXW), jnp.int32),
            pltpu.VMEM((ROWS_W, IDXW), jnp.int32),
            pltpu.VMEM((DROWS_W, IDXW), jnp.int32),
            pltpu.VMEM((13, IDXW), jnp.int32),
            pltpu.VMEM((2 * K * IDXW, D), jnp.float32),
            pltpu.SemaphoreType.DMA,
            pltpu.SemaphoreType.DMA,
        ],
        compiler_params=pltpu.CompilerParams(use_tc_tiling_on_sc=False),
    )
    out = sc(tbl, sidx, dtok, jnp.asarray(_OIDX_S), jnp.asarray(_OIDX_D),
             jnp.asarray(_OFFS))
    return out.reshape(B, NF, D)


def kernel(sparse_inputs, dense_inputs, tables, W):
    return _tokenize(sparse_inputs, dense_inputs, tables, W)


# speedup vs baseline: 1.3700x; 1.3700x over previous
import functools

import jax
import jax.numpy as jnp
from jax import lax
from jax.experimental import pallas as pl
from jax.experimental.pallas import tpu as pltpu
from jax.experimental.pallas import tpu_sc as plsc

B = 16384
F = 26
V = 100000
D = 32
ND = 13
NF = F + ND

NC = 2
NS = 16
NW = NC * NS

RPF = B // 128


def _mm_body(xt_ref, wt_ref, o_ref):
    o_ref[...] = jnp.dot(wt_ref[...], xt_ref[...],
                         preferred_element_type=jnp.float32)


def _dense_proj_t(xt, wt):
    nb = 2048
    return pl.pallas_call(
        _mm_body,
        grid=(B // nb,),
        in_specs=[pl.BlockSpec((ND, nb), lambda i: (0, i)),
                  pl.BlockSpec((ND * D, ND), lambda i: (0, 0))],
        out_specs=pl.BlockSpec((ND * D, nb), lambda i: (0, i)),
        out_shape=jax.ShapeDtypeStruct((ND * D, B), jnp.float32),
    )(xt, wt)


def _sc_body(tbl_hbm, sidx_hbm, dtok_hbm, out_hbm,
             idxv, gbuf, dbuf, gsem, ssem, dsem):
    wid = lax.axis_index("s") * NC + lax.axis_index("c")

    def field_body(f, carry):
        goff = lax.rem(f, 2) * B
        slab = tbl_hbm.at[f * D + wid]

        @pl.when(f >= 2)
        def _():
            pltpu.make_async_copy(gbuf.at[pl.ds(goff, B)],
                                  out_hbm.at[0], ssem).wait()

        pltpu.sync_copy(sidx_hbm.at[f], idxv)

        def gblk(r8, c):
            for j in range(8):
                r = r8 * 8 + j
                pltpu.async_copy(slab.at[idxv.at[r]],
                                 gbuf.at[pl.ds(goff + r * 128, 128)], gsem)
            return c
        lax.fori_loop(0, RPF // 8, gblk, 0)
        pltpu.make_async_copy(slab.at[pl.ds(0, B)],
                              gbuf.at[pl.ds(goff, B)], gsem).wait()

        pltpu.async_copy(gbuf.at[pl.ds(goff, B)],
                         out_hbm.at[f * D + wid], ssem)
        return carry
    lax.fori_loop(0, F, field_body, 0)

    def dense_body(j, carry):
        doff = lax.rem(j, 2) * B

        @pl.when(j >= 2)
        def _():
            pltpu.make_async_copy(dbuf.at[pl.ds(doff, B)],
                                  out_hbm.at[0], dsem).wait()

        pltpu.async_copy(dtok_hbm.at[j * D + wid],
                         dbuf.at[pl.ds(doff, B)], gsem)
        pltpu.make_async_copy(dtok_hbm.at[0],
                              dbuf.at[pl.ds(doff, B)], gsem).wait()
        pltpu.async_copy(dbuf.at[pl.ds(doff, B)],
                         out_hbm.at[(F + j) * D + wid], dsem)
        return carry
    lax.fori_loop(0, ND, dense_body, 0)

    def drain(sem):
        pltpu.make_async_copy(gbuf.at[pl.ds(0, B)], out_hbm.at[0], sem).wait()
    drain(ssem)
    drain(ssem)
    drain(dsem)
    drain(dsem)


@jax.jit
def _tokenize(sparse_inputs, dense_inputs, tables, w):
    tbl_t = tables.transpose(0, 2, 1).reshape(F * D, V)
    sidx_t = sparse_inputs.T.reshape(F, RPF, 128)
    dtok_t = _dense_proj_t(dense_inputs.T, w.T)

    mesh = plsc.VectorSubcoreMesh(core_axis_name="c", subcore_axis_name="s",
                                  num_cores=NC, num_subcores=NS)
    sc = pl.kernel(
        _sc_body,
        out_type=jax.ShapeDtypeStruct((NF * D, B), jnp.float32),
        mesh=mesh,
        scratch_types=[
            pltpu.VMEM((RPF, 128), jnp.int32),
            pltpu.VMEM((2 * B,), jnp.float32),
            pltpu.VMEM((2 * B,), jnp.float32),
            pltpu.SemaphoreType.DMA,
            pltpu.SemaphoreType.DMA,
            pltpu.SemaphoreType.DMA,
        ],
        compiler_params=pltpu.CompilerParams(use_tc_tiling_on_sc=False),
    )
    out = sc(tbl_t, sidx_t, dtok_t)
    return out.reshape(NF, D, B).transpose(2, 0, 1)


def kernel(sparse_inputs, dense_inputs, tables, W):
    return _tokenize(sparse_inputs, dense_inputs, tables, W)


# speedup vs baseline: 1.4422x; 1.0528x over previous
import functools

import jax
import jax.numpy as jnp
from jax import lax
from jax.experimental import pallas as pl
from jax.experimental.pallas import tpu as pltpu
from jax.experimental.pallas import tpu_sc as plsc

B = 16384
F = 26
V = 100000
D = 32
ND = 13
NF = F + ND

NC = 2
NS = 16
NW = NC * NS

RPF = B // 128


def _mm_body(xt_ref, wt_ref, o_ref):
    o_ref[...] = jnp.dot(wt_ref[...], xt_ref[...],
                         preferred_element_type=jnp.float32)


def _dense_proj_t(xt, wt):
    nb = 2048
    return pl.pallas_call(
        _mm_body,
        grid=(B // nb,),
        in_specs=[pl.BlockSpec((ND, nb), lambda i: (0, i)),
                  pl.BlockSpec((ND * D, ND), lambda i: (0, 0))],
        out_specs=pl.BlockSpec((ND * D, nb), lambda i: (0, i)),
        out_shape=jax.ShapeDtypeStruct((ND * D, B), jnp.float32),
    )(xt, wt)


def _sc_body(tbl_hbm, sidx_hbm, dtok_hbm, out_hbm,
             idxv, gbuf, dbuf, isem, gsemA, gsemB, ssem, dlsem, dsem):
    wid = lax.axis_index("s") * NC + lax.axis_index("c")

    def fire_gathers(f, par, gsem):
        goff = par * B
        slab = tbl_hbm.at[f * D + wid]

        def gblk(r8, c):
            for j in range(8):
                r = r8 * 8 + j
                pltpu.async_copy(slab.at[idxv.at[par, r]],
                                 gbuf.at[pl.ds(goff + r * 128, 128)], gsem)
            return c
        lax.fori_loop(0, RPF // 8, gblk, 0)

    def drain_gathers(par, gsem):
        pltpu.make_async_copy(tbl_hbm.at[0].at[pl.ds(0, B)],
                              gbuf.at[pl.ds(par * B, B)], gsem).wait()

    def drain_write(sem):
        pltpu.make_async_copy(gbuf.at[pl.ds(0, B)], out_hbm.at[0], sem).wait()

    def drain_idx():
        pltpu.make_async_copy(sidx_hbm.at[0], idxv.at[0], isem).wait()

    pltpu.sync_copy(sidx_hbm.at[0], idxv.at[0])
    fire_gathers(0, 0, gsemA)
    pltpu.async_copy(sidx_hbm.at[1], idxv.at[1], isem)

    def field_body(f, carry):
        par = lax.rem(f, 2)

        drain_idx()

        @pl.when(f >= 2)
        def _():
            drain_write(ssem)

        @pl.when(par == 0)
        def _():
            fire_gathers(f, 0, gsemA)
            drain_gathers(1, gsemB)

        @pl.when(par == 1)
        def _():
            fire_gathers(f, 1, gsemB)
            drain_gathers(0, gsemA)

        pltpu.async_copy(gbuf.at[pl.ds((1 - par) * B, B)],
                         out_hbm.at[(f - 1) * D + wid], ssem)

        @pl.when(f < F - 1)
        def _():
            pltpu.async_copy(sidx_hbm.at[f + 1], idxv.at[1 - par], isem)

        jd = f - 2

        @pl.when((jd >= 0) & (jd < ND))
        def _():
            @pl.when(jd >= 2)
            def _():
                drain_write(dsem)
            pltpu.async_copy(dtok_hbm.at[jd * D + wid],
                             dbuf.at[pl.ds(lax.rem(jd, 2) * B, B)], dlsem)

            @pl.when(jd >= 1)
            def _():
                drain_write(dlsem)
                pltpu.async_copy(dbuf.at[pl.ds(lax.rem(jd - 1, 2) * B, B)],
                                 out_hbm.at[(F + jd - 1) * D + wid], dsem)
        return carry
    lax.fori_loop(1, F, field_body, 0)

    drain_gathers(1, gsemB)
    pltpu.async_copy(gbuf.at[pl.ds(B, B)], out_hbm.at[(F - 1) * D + wid], ssem)
    drain_write(dlsem)
    pltpu.async_copy(dbuf.at[pl.ds(0, B)],
                     out_hbm.at[(F + ND - 1) * D + wid], dsem)
    drain_write(ssem)
    drain_write(ssem)
    drain_write(dsem)
    drain_write(dsem)


@jax.jit
def _tokenize(sparse_inputs, dense_inputs, tables, w):
    tbl_t = tables.transpose(0, 2, 1).reshape(F * D, V)
    sidx_t = sparse_inputs.T.reshape(F, RPF, 128)
    dtok_t = _dense_proj_t(dense_inputs.T, w.T)

    mesh = plsc.VectorSubcoreMesh(core_axis_name="c", subcore_axis_name="s",
                                  num_cores=NC, num_subcores=NS)
    sc = pl.kernel(
        _sc_body,
        out_type=jax.ShapeDtypeStruct((NF * D, B), jnp.float32),
        mesh=mesh,
        scratch_types=[
            pltpu.VMEM((2, RPF, 128), jnp.int32),
            pltpu.VMEM((2 * B,), jnp.float32),
            pltpu.VMEM((2 * B,), jnp.float32),
            pltpu.SemaphoreType.DMA,
            pltpu.SemaphoreType.DMA,
            pltpu.SemaphoreType.DMA,
            pltpu.SemaphoreType.DMA,
            pltpu.SemaphoreType.DMA,
            pltpu.SemaphoreType.DMA,
        ],
        compiler_params=pltpu.CompilerParams(use_tc_tiling_on_sc=False),
    )
    out = sc(tbl_t, sidx_t, dtok_t)
    return out.reshape(NF, D, B).transpose(2, 0, 1)


def kernel(sparse_inputs, dense_inputs, tables, W):
    return _tokenize(sparse_inputs, dense_inputs, tables, W)
